# baseline (speedup 1.0000x reference)
import functools

import jax
import jax.numpy as jnp
from jax import lax
from jax.experimental import pallas as pl
from jax.experimental.pallas import tpu as pltpu
from jax.experimental.pallas import tpu_sc as plsc

N = 64
CH = 30
CELLS = 14 * 14
NC, NS, L = 2, 16, 16
NW = NC * NS
BG = N // L
TASKS = CELLS * BG
BASE_T = TASKS // NW
EXTRA = TASKS - BASE_T * NW
CSPAN = 7
INV14 = 1.0 / 14.0


def _sqrtf(x):
    i = lax.bitcast_convert_type(x, jnp.int32)
    y = lax.bitcast_convert_type(0x5F3759DF - (i >> 1), jnp.float32)
    xh = 0.5 * x
    for _ in range(3):
        y = y * (1.5 - xh * y * y)
    return x * y


def _contrib(pv, tv, cl, b0):
    def g(ref, c):
        return ref[cl, c, pl.ds(b0, L)]

    t4 = g(tv, 4)
    coo = t4 > 0.0
    noo = t4 == 0.0
    p4 = g(pv, 4)
    p9 = g(pv, 9)
    t9 = g(tv, 9)
    d4 = p4 - t4
    d9 = p9 - t9
    noobj = d4 * d4 + d9 * d9

    cls = None
    for c in range(10, 30):
        d = g(pv, c) - g(tv, c)
        cls = d * d if cls is None else cls + d * d

    p0, p1, p2, p3 = g(pv, 0), g(pv, 1), g(pv, 2), g(pv, 3)
    p5, p6, p7, p8 = g(pv, 5), g(pv, 6), g(pv, 7), g(pv, 8)
    t0, t1, t2, t3 = g(tv, 0), g(tv, 1), g(tv, 2), g(tv, 3)
    t5, t6, t7, t8 = g(tv, 5), g(tv, 6), g(tv, 7), g(tv, 8)

    tltx = t0 * INV14 - 0.5 * t2
    trbx = t0 * INV14 + 0.5 * t2
    tlty = t1 * INV14 - 0.5 * t3
    trby = t1 * INV14 + 0.5 * t3
    area2 = (trbx - tltx) * (trby - tlty)

    def iou_parts(cx, cy, w, h):
        ltx = cx * INV14 - 0.5 * w
        rbx = cx * INV14 + 0.5 * w
        lty = cy * INV14 - 0.5 * h
        rby = cy * INV14 + 0.5 * h
        wx = jnp.maximum(jnp.minimum(rbx, trbx) - jnp.maximum(ltx, tltx), 0.0)
        wy = jnp.maximum(jnp.minimum(rby, trby) - jnp.maximum(lty, tlty), 0.0)
        inter = wx * wy
        a1 = (rbx - ltx) * (rby - lty)
        return inter, a1 + area2 - inter

    inter0, den0 = iou_parts(p0, p1, p2, p3)
    inter1, den1 = iou_parts(p5, p6, p7, p8)
    sel0 = inter0 * den1 >= inter1 * den0

    rx = jnp.where(sel0, p0, p5)
    ry = jnp.where(sel0, p1, p6)
    rw = jnp.where(sel0, p2, p7)
    rh = jnp.where(sel0, p3, p8)
    rc = jnp.where(sel0, p4, p9)
    oc = jnp.where(sel0, p9, p4)
    sx = jnp.where(sel0, t0, t5)
    sy = jnp.where(sel0, t1, t6)
    sw = jnp.where(sel0, t2, t7)
    sh = jnp.where(sel0, t3, t8)

    dx = rx - sx
    dy = ry - sy
    dw = _sqrtf(rw) - _sqrtf(sw)
    dh = _sqrtf(rh) - _sqrtf(sh)
    loc = dx * dx + dy * dy + dw * dw + dh * dh

    obj = 5.0 * loc + 2.0 * (rc * rc) + oc * oc + cls
    return jnp.where(coo, obj, 0.0) + jnp.where(noo, 0.5 * noobj, 0.0)


def _body(pred_hbm, target_hbm, out_hbm, pv, tv, acc_v, sem_p, sem_t, sem_p2, sem_t2):
    wid = lax.axis_index("s") * NC + lax.axis_index("c")
    start = wid * BASE_T + jnp.minimum(wid, EXTRA)
    cnt = BASE_T + jnp.where(wid < EXTRA, 1, 0)
    kbase = jnp.minimum(start // BG, CELLS - CSPAN)
    HEAD = 1
    cph = pltpu.async_copy(pred_hbm.at[pl.ds(kbase, HEAD)], pv.at[pl.ds(0, HEAD)], sem_p)
    cth = pltpu.async_copy(target_hbm.at[pl.ds(kbase, HEAD)], tv.at[pl.ds(0, HEAD)], sem_t)
    cpt = pltpu.async_copy(
        pred_hbm.at[pl.ds(kbase + HEAD, CSPAN - HEAD)], pv.at[pl.ds(HEAD, CSPAN - HEAD)], sem_p2)
    ctt = pltpu.async_copy(
        target_hbm.at[pl.ds(kbase + HEAD, CSPAN - HEAD)], tv.at[pl.ds(HEAD, CSPAN - HEAD)], sem_t2)
    thead = (kbase + HEAD) * BG - start

    def step(i, acc):
        @pl.when(i == 0)
        def _():
            cph.wait()
            cth.wait()

        @pl.when(i == thead)
        def _():
            cpt.wait()
            ctt.wait()

        t = start + i
        k = t // BG
        b0 = (t - k * BG) * L
        return acc + _contrib(pv, tv, k - kbase, b0)

    acc = lax.fori_loop(0, cnt, step, jnp.zeros((L,), jnp.float32))

    acc_v[...] = acc
    pltpu.sync_copy(acc_v, out_hbm.at[wid])


@jax.jit
def kernel(pred, target):
    pt = jnp.transpose(pred, (1, 2, 3, 0)).reshape(CELLS, CH, N)
    tt = jnp.transpose(target, (1, 2, 3, 0)).reshape(CELLS, CH, N)
    partials = pl.kernel(
        _body,
        out_type=jax.ShapeDtypeStruct((NW, L), jnp.float32),
        mesh=plsc.VectorSubcoreMesh(core_axis_name="c", subcore_axis_name="s"),
        scratch_types=[
            pltpu.VMEM((CSPAN, CH, N), jnp.float32),
            pltpu.VMEM((CSPAN, CH, N), jnp.float32),
            pltpu.VMEM((L,), jnp.float32),
            pltpu.SemaphoreType.DMA,
            pltpu.SemaphoreType.DMA,
            pltpu.SemaphoreType.DMA,
            pltpu.SemaphoreType.DMA,
        ],
    )(pt, tt)
    return jnp.sum(partials) / N


# speedup vs baseline: 1.0117x; 1.0117x over previous
import functools

import jax
import jax.numpy as jnp
from jax import lax
from jax.experimental import pallas as pl
from jax.experimental.pallas import tpu as pltpu
from jax.experimental.pallas import tpu_sc as plsc

N = 64
CH = 30
CELLS = 14 * 14
NC, NS, L = 2, 16, 16
NW = NC * NS
BG = N // L
TASKS = CELLS * BG
BASE_T = TASKS // NW
EXTRA = TASKS - BASE_T * NW
CSPAN = 7
INV14 = 1.0 / 14.0


def _sqrtf(x):
    i = lax.bitcast_convert_type(x, jnp.int32)
    y = lax.bitcast_convert_type(0x5F3759DF - (i >> 1), jnp.float32)
    xh = 0.5 * x
    for _ in range(3):
        y = y * (1.5 - xh * y * y)
    return x * y


def _contrib(pv, tv, cl, b0):
    def g(ref, c):
        return ref[cl, c, pl.ds(b0, L)]

    t4 = g(tv, 4)
    coo = t4 > 0.0
    noo = t4 == 0.0
    p4 = g(pv, 4)
    p9 = g(pv, 9)
    t9 = g(tv, 9)
    d4 = p4 - t4
    d9 = p9 - t9
    noobj = d4 * d4 + d9 * d9

    cls = None
    for c in range(10, 30):
        d = g(pv, c) - g(tv, c)
        cls = d * d if cls is None else cls + d * d

    p0, p1, p2, p3 = g(pv, 0), g(pv, 1), g(pv, 2), g(pv, 3)
    p5, p6, p7, p8 = g(pv, 5), g(pv, 6), g(pv, 7), g(pv, 8)
    t0, t1, t2, t3 = g(tv, 0), g(tv, 1), g(tv, 2), g(tv, 3)
    t5, t6, t7, t8 = g(tv, 5), g(tv, 6), g(tv, 7), g(tv, 8)

    tltx = t0 * INV14 - 0.5 * t2
    trbx = t0 * INV14 + 0.5 * t2
    tlty = t1 * INV14 - 0.5 * t3
    trby = t1 * INV14 + 0.5 * t3
    area2 = (trbx - tltx) * (trby - tlty)

    def iou_parts(cx, cy, w, h):
        ltx = cx * INV14 - 0.5 * w
        rbx = cx * INV14 + 0.5 * w
        lty = cy * INV14 - 0.5 * h
        rby = cy * INV14 + 0.5 * h
        wx = jnp.maximum(jnp.minimum(rbx, trbx) - jnp.maximum(ltx, tltx), 0.0)
        wy = jnp.maximum(jnp.minimum(rby, trby) - jnp.maximum(lty, tlty), 0.0)
        inter = wx * wy
        a1 = (rbx - ltx) * (rby - lty)
        return inter, a1 + area2 - inter

    inter0, den0 = iou_parts(p0, p1, p2, p3)
    inter1, den1 = iou_parts(p5, p6, p7, p8)
    sel0 = inter0 * den1 >= inter1 * den0

    rx = jnp.where(sel0, p0, p5)
    ry = jnp.where(sel0, p1, p6)
    rw = jnp.where(sel0, p2, p7)
    rh = jnp.where(sel0, p3, p8)
    rc = jnp.where(sel0, p4, p9)
    oc = jnp.where(sel0, p9, p4)
    sx = jnp.where(sel0, t0, t5)
    sy = jnp.where(sel0, t1, t6)
    sw = jnp.where(sel0, t2, t7)
    sh = jnp.where(sel0, t3, t8)

    dx = rx - sx
    dy = ry - sy
    wh2 = rw + sw - 2.0 * _sqrtf(rw * sw) + rh + sh - 2.0 * _sqrtf(rh * sh)
    loc = dx * dx + dy * dy + wh2

    obj = 5.0 * loc + 2.0 * (rc * rc) + oc * oc + cls
    return jnp.where(coo, obj, 0.0) + jnp.where(noo, 0.5 * noobj, 0.0)


def _body(pred_hbm, target_hbm, out_hbm, pv, tv, acc_v, sem_p, sem_t, sem_p2, sem_t2):
    wid = lax.axis_index("s") * NC + lax.axis_index("c")
    start = wid * BASE_T + jnp.minimum(wid, EXTRA)
    cnt = BASE_T + jnp.where(wid < EXTRA, 1, 0)
    kbase = jnp.minimum(start // BG, CELLS - CSPAN)
    HEAD = 2
    cph = pltpu.async_copy(pred_hbm.at[pl.ds(kbase, HEAD)], pv.at[pl.ds(0, HEAD)], sem_p)
    cth = pltpu.async_copy(target_hbm.at[pl.ds(kbase, HEAD)], tv.at[pl.ds(0, HEAD)], sem_t)
    cpt = pltpu.async_copy(
        pred_hbm.at[pl.ds(kbase + HEAD, CSPAN - HEAD)], pv.at[pl.ds(HEAD, CSPAN - HEAD)], sem_p2)
    ctt = pltpu.async_copy(
        target_hbm.at[pl.ds(kbase + HEAD, CSPAN - HEAD)], tv.at[pl.ds(HEAD, CSPAN - HEAD)], sem_t2)
    thead = (kbase + HEAD) * BG - start

    def step(i, acc):
        @pl.when(i == 0)
        def _():
            cph.wait()
            cth.wait()

        @pl.when(i == thead)
        def _():
            cpt.wait()
            ctt.wait()

        t = start + i
        k = t // BG
        b0 = (t - k * BG) * L
        return acc + _contrib(pv, tv, k - kbase, b0)

    acc = lax.fori_loop(0, cnt, step, jnp.zeros((L,), jnp.float32))

    acc_v[...] = acc
    pltpu.sync_copy(acc_v, out_hbm.at[wid])


@jax.jit
def kernel(pred, target):
    pt = jnp.transpose(pred, (1, 2, 3, 0)).reshape(CELLS, CH, N)
    tt = jnp.transpose(target, (1, 2, 3, 0)).reshape(CELLS, CH, N)
    partials = pl.kernel(
        _body,
        out_type=jax.ShapeDtypeStruct((NW, L), jnp.float32),
        mesh=plsc.VectorSubcoreMesh(core_axis_name="c", subcore_axis_name="s"),
        scratch_types=[
            pltpu.VMEM((CSPAN, CH, N), jnp.float32),
            pltpu.VMEM((CSPAN, CH, N), jnp.float32),
            pltpu.VMEM((L,), jnp.float32),
            pltpu.SemaphoreType.DMA,
            pltpu.SemaphoreType.DMA,
            pltpu.SemaphoreType.DMA,
            pltpu.SemaphoreType.DMA,
        ],
    )(pt, tt)
    return jnp.sum(partials) / N
